# baseline (speedup 1.0000x reference)
import functools

import jax
import jax.numpy as jnp
from jax import lax
from jax.experimental import pallas as pl
from jax.experimental.pallas import tpu as pltpu
from jax.experimental.pallas import tpu_sc as plsc

N = 10000
N_PAD = 10240
E = 320000
D = 128

NC = 2
NS = 16
NW = NC * NS
EPW = E // NW
CHUNK = 80
NCHUNK = 126
EPP = NCHUNK * CHUNK
RPT = N_PAD // NS

_mesh = plsc.VectorSubcoreMesh(core_axis_name="c", subcore_axis_name="s")
_sc_params = pltpu.CompilerParams(needs_layout_passes=False)


@functools.partial(
    pl.kernel,
    mesh=_mesh,
    out_type=(jax.ShapeDtypeStruct((NW, N_PAD), jnp.float32),
              jax.ShapeDtypeStruct((NC, N_PAD), jnp.float32)),
    compiler_params=_sc_params,
    scratch_types=[
        pltpu.VMEM((EPW,), jnp.int32),
        pltpu.VMEM((N_PAD,), jnp.float32),
        pltpu.VMEM((NS, RPT), jnp.float32),
        pltpu.SemaphoreType.DMA,
    ],
)
def _deg_kernel(dst_hbm, part_hbm, red_hbm, dst_v, acc_v, red_v, sem):
    c = lax.axis_index("c")
    s = lax.axis_index("s")
    w = c * NS + s

    icp = pltpu.async_copy(dst_hbm.at[pl.ds(w * EPW, EPW)], dst_v, sem)

    zeros16 = jnp.zeros((16,), jnp.float32)
    ones16 = jnp.ones((16,), jnp.float32)

    def zero_body(i, _):
        acc_v[pl.ds(i * 16, 16)] = zeros16
        return 0

    lax.fori_loop(0, N_PAD // 16, zero_body, 0)
    icp.wait()

    def count_body(j, _):
        idx = dst_v[pl.ds(j * 16, 16)]
        plsc.addupdate_scatter(acc_v, [idx], ones16)
        return 0

    lax.fori_loop(0, EPW // 16, count_body, 0)
    pltpu.sync_copy(acc_v, part_hbm.at[w])
    plsc.subcore_barrier()

    pltpu.sync_copy(part_hbm.at[pl.ds(c * NS, NS), pl.ds(s * RPT, RPT)], red_v)

    def red_body(g, _):
        v = zeros16
        for r in range(NS):
            v = v + red_v[r, pl.ds(g * 16, 16)]
        acc_v[pl.ds(g * 16, 16)] = v
        return 0

    lax.fori_loop(0, RPT // 16, red_body, 0)
    pltpu.sync_copy(acc_v.at[pl.ds(0, RPT)], red_hbm.at[c, pl.ds(s * RPT, RPT)])


@functools.partial(
    pl.kernel,
    mesh=_mesh,
    out_type=jax.ShapeDtypeStruct((NC, N_PAD, D), jnp.float32),
    scratch_types=[
        pltpu.VMEM((EPP,), jnp.int32),
        [pltpu.VMEM((CHUNK,), jnp.int32) for _ in range(2)],
        [pltpu.VMEM((CHUNK, D), jnp.float32) for _ in range(2)],
        pltpu.VMEM((128, D), jnp.float32),
        pltpu.VMEM_SHARED((N_PAD, D), jnp.float32),
        pltpu.SemaphoreType.DMA,
        [pltpu.SemaphoreType.DMA for _ in range(2)],
        [pltpu.SemaphoreType.DMA for _ in range(2)],
    ],
)
def _agg_kernel(u_hbm, src_hbm, dst_hbm, out_hbm,
                src_v, dstb, rows, buf_v, acc_sh, isem, gsems, ssems):
    c = lax.axis_index("c")
    s = lax.axis_index("s")
    w = c * NS + s

    icp = pltpu.async_copy(src_hbm.at[pl.ds(w * EPP, EPP)], src_v, isem)

    zeros16 = jnp.zeros((16,), jnp.float32)

    def zbuf_body(i, _):
        for j in range(D // 16):
            buf_v[i, pl.ds(j * 16, 16)] = zeros16
        return 0

    lax.fori_loop(0, 128, zbuf_body, 0)

    for i in range(RPT // 128):
        pltpu.sync_copy(buf_v, acc_sh.at[pl.ds(s * RPT + i * 128, 128)])
    icp.wait()
    plsc.subcore_barrier()

    def gather_start(j, b):
        base = pl.multiple_of(w * EPP + j * CHUNK, 8)
        off = pl.multiple_of(j * CHUNK, 8)
        pltpu.async_copy(dst_hbm.at[pl.ds(base, CHUNK)], dstb[b], gsems[b])
        pltpu.async_copy(
            u_hbm.at[src_v.at[pl.ds(off, CHUNK)]], rows[b], gsems[b])

    def gather_wait(b):
        pltpu.make_async_copy(
            dst_hbm.at[pl.ds(0, CHUNK)], dstb[b], gsems[b]).wait()
        pltpu.make_async_copy(
            u_hbm.at[src_v.at[pl.ds(0, CHUNK)]], rows[b], gsems[b]).wait()

    def scatter_start(b):
        pltpu.async_copy(rows[b], acc_sh.at[dstb[b]], ssems[b], add=True)

    def scatter_wait(b):
        pltpu.make_async_copy(rows[b], acc_sh.at[dstb[b]], ssems[b]).wait()

    gather_start(0, 0)
    gather_wait(0)
    scatter_start(0)
    gather_start(1, 1)
    gather_wait(1)
    scatter_start(1)
    scatter_wait(0)
    gather_start(2, 0)

    def edge_body(g, _):
        j = 2 * g
        gather_wait(0)
        scatter_start(0)
        scatter_wait(1)
        gather_start(j + 1, 1)
        gather_wait(1)
        scatter_start(1)
        scatter_wait(0)
        gather_start(j + 2, 0)
        return 0

    lax.fori_loop(1, NCHUNK // 2 - 1, edge_body, 0)
    gather_wait(0)
    scatter_start(0)
    scatter_wait(1)
    gather_start(NCHUNK - 1, 1)
    gather_wait(1)
    scatter_start(1)
    scatter_wait(0)
    scatter_wait(1)
    plsc.subcore_barrier()

    for i in range(RPT // 128):
        pltpu.sync_copy(acc_sh.at[pl.ds(s * RPT + i * 128, 128)], buf_v)
        pltpu.sync_copy(buf_v, out_hbm.at[c, pl.ds(s * RPT + i * 128, 128)])


_BLK = 2048
_GRID = N_PAD // _BLK


def _dinv(degp_ref):
    deg = jnp.sum(degp_ref[...], axis=0) + 1.0
    return lax.rsqrt(deg)


def _first_body(x_ref, w_ref, degp_ref, out_ref):
    dinv = _dinv(degp_ref)
    out_ref[...] = jnp.dot(x_ref[...], w_ref[...],
                           preferred_element_type=jnp.float32) * dinv


def _mid_body(agg_ref, u_ref, b_ref, w_ref, degp_ref, out_ref):
    dinv = _dinv(degp_ref)
    t = agg_ref[0] + agg_ref[1] + u_ref[...]
    h = jnp.maximum(t * dinv + b_ref[...], 0.0)
    out_ref[...] = jnp.dot(h, w_ref[...],
                           preferred_element_type=jnp.float32) * dinv


def _last_body(agg_ref, u_ref, b_ref, degp_ref, out_ref):
    i = pl.program_id(0)
    dinv = _dinv(degp_ref)
    t = agg_ref[0] + agg_ref[1] + u_ref[...]
    h = jnp.maximum(t * dinv + b_ref[...], 0.0)
    row = lax.broadcasted_iota(jnp.int32, (_BLK, 1), 0) + i * _BLK
    h = jnp.where(row < N, h, 0.0)
    part = jnp.sum(h, axis=0, keepdims=True) * (1.0 / N)

    @pl.when(i == 0)
    def _():
        out_ref[...] = jnp.zeros_like(out_ref)

    out_ref[...] += part


_rows_spec = pl.BlockSpec((_BLK, D), lambda i: (i, 0))
_w_spec = pl.BlockSpec((D, D), lambda i: (0, 0))
_b_spec = pl.BlockSpec((1, D), lambda i: (0, 0))
_degp_spec = pl.BlockSpec((NC, _BLK, 1), lambda i: (0, i, 0))
_agg_spec = pl.BlockSpec((NC, _BLK, D), lambda i: (0, i, 0))

_first_tc = pl.pallas_call(
    _first_body,
    grid=(_GRID,),
    in_specs=[_rows_spec, _w_spec, _degp_spec],
    out_specs=_rows_spec,
    out_shape=jax.ShapeDtypeStruct((N_PAD, D), jnp.float32),
)

_mid_tc = pl.pallas_call(
    _mid_body,
    grid=(_GRID,),
    in_specs=[_agg_spec, _rows_spec, _b_spec, _w_spec, _degp_spec],
    out_specs=_rows_spec,
    out_shape=jax.ShapeDtypeStruct((N_PAD, D), jnp.float32),
)

_last_tc = pl.pallas_call(
    _last_body,
    grid=(_GRID,),
    in_specs=[_agg_spec, _rows_spec, _b_spec, _degp_spec],
    out_specs=pl.BlockSpec((1, D), lambda i: (0, 0)),
    out_shape=jax.ShapeDtypeStruct((1, D), jnp.float32),
)


def kernel(x, edge_index, W1, b1, W2, b2, W3, b3):
    src = edge_index[0]
    dst = edge_index[1]
    src_p = jnp.pad(src.reshape(NW, EPW), ((0, 0), (0, EPP - EPW))).reshape(-1)
    dst_p = jnp.pad(dst.reshape(NW, EPW), ((0, 0), (0, EPP - EPW)),
                    constant_values=N_PAD - 1).reshape(-1)
    x_pad = jnp.pad(x, ((0, N_PAD - N), (0, 0)))
    b1r = b1.reshape(1, D)
    b2r = b2.reshape(1, D)
    b3r = b3.reshape(1, D)

    _, degr = _deg_kernel(dst)
    degp3 = degr.reshape(NC, N_PAD, 1)

    u1 = _first_tc(x_pad, W1, degp3)
    agg1 = _agg_kernel(u1, src_p, dst_p)
    u2 = _mid_tc(agg1, u1, b1r, W2, degp3)
    agg2 = _agg_kernel(u2, src_p, dst_p)
    u3 = _mid_tc(agg2, u2, b2r, W3, degp3)
    agg3 = _agg_kernel(u3, src_p, dst_p)
    out = _last_tc(agg3, u3, b3r, degp3)
    return out.reshape(D)


# speedup vs baseline: 1.0030x; 1.0030x over previous
import functools

import jax
import jax.numpy as jnp
from jax import lax
from jax.experimental import pallas as pl
from jax.experimental.pallas import tpu as pltpu
from jax.experimental.pallas import tpu_sc as plsc

N = 10000
N_PAD = 10240
E = 320000
D = 128

NC = 2
NS = 16
NW = NC * NS
EPW = E // NW
CHUNK = 80
NCHUNK = 126
EPP = NCHUNK * CHUNK
RPT = N_PAD // NS

_mesh = plsc.VectorSubcoreMesh(core_axis_name="c", subcore_axis_name="s")
_sc_params = pltpu.CompilerParams(needs_layout_passes=False)


@functools.partial(
    pl.kernel,
    mesh=_mesh,
    out_type=jax.ShapeDtypeStruct((NC, N_PAD), jnp.float32),
    compiler_params=_sc_params,
    scratch_types=[
        pltpu.VMEM((EPW,), jnp.int32),
        pltpu.VMEM((N_PAD,), jnp.float32),
        pltpu.VMEM((NS, RPT), jnp.float32),
        pltpu.VMEM_SHARED((NS, N_PAD), jnp.float32),
        pltpu.SemaphoreType.DMA,
    ],
)
def _deg_kernel(dst_hbm, out_hbm, dst_v, acc_v, red_v, stage_sh, sem):
    c = lax.axis_index("c")
    s = lax.axis_index("s")
    w = c * NS + s

    icp = pltpu.async_copy(dst_hbm.at[pl.ds(w * EPW, EPW)], dst_v, sem)

    zeros16 = jnp.zeros((16,), jnp.float32)
    ones16 = jnp.ones((16,), jnp.float32)

    def zero_body(i, _):
        acc_v[pl.ds(i * 16, 16)] = zeros16
        return 0

    lax.fori_loop(0, N_PAD // 16, zero_body, 0)
    icp.wait()

    def count_body(j, _):
        idx = dst_v[pl.ds(j * 16, 16)]
        plsc.addupdate_scatter(acc_v, [idx], ones16)
        return 0

    lax.fori_loop(0, EPW // 16, count_body, 0)

    pltpu.sync_copy(acc_v, stage_sh.at[s])
    plsc.subcore_barrier()
    pltpu.sync_copy(stage_sh.at[:, pl.ds(s * RPT, RPT)], red_v)

    def red_body(g, _):
        v = zeros16
        for r in range(NS):
            v = v + red_v[r, pl.ds(g * 16, 16)]
        acc_v[pl.ds(g * 16, 16)] = v
        return 0

    lax.fori_loop(0, RPT // 16, red_body, 0)
    pltpu.sync_copy(acc_v.at[pl.ds(0, RPT)], out_hbm.at[c, pl.ds(s * RPT, RPT)])


@functools.partial(
    pl.kernel,
    mesh=_mesh,
    out_type=jax.ShapeDtypeStruct((NC, N_PAD, D), jnp.float32),
    scratch_types=[
        pltpu.VMEM((EPP,), jnp.int32),
        [pltpu.VMEM((CHUNK,), jnp.int32) for _ in range(2)],
        [pltpu.VMEM((CHUNK, D), jnp.float32) for _ in range(2)],
        pltpu.VMEM((128, D), jnp.float32),
        pltpu.VMEM_SHARED((N_PAD, D), jnp.float32),
        pltpu.SemaphoreType.DMA,
        [pltpu.SemaphoreType.DMA for _ in range(2)],
        [pltpu.SemaphoreType.DMA for _ in range(2)],
    ],
)
def _agg_kernel(u_hbm, src_hbm, dst_hbm, out_hbm,
                src_v, dstb, rows, buf_v, acc_sh, isem, gsems, ssems):
    c = lax.axis_index("c")
    s = lax.axis_index("s")
    w = c * NS + s

    icp = pltpu.async_copy(src_hbm.at[pl.ds(w * EPP, EPP)], src_v, isem)

    zeros16 = jnp.zeros((16,), jnp.float32)

    def zbuf_body(i, _):
        for j in range(D // 16):
            buf_v[i, pl.ds(j * 16, 16)] = zeros16
        return 0

    lax.fori_loop(0, 128, zbuf_body, 0)

    for i in range(RPT // 128):
        pltpu.sync_copy(buf_v, acc_sh.at[pl.ds(s * RPT + i * 128, 128)])
    icp.wait()
    plsc.subcore_barrier()

    def gather_start(j, b):
        base = pl.multiple_of(w * EPP + j * CHUNK, 8)
        off = pl.multiple_of(j * CHUNK, 8)
        pltpu.async_copy(dst_hbm.at[pl.ds(base, CHUNK)], dstb[b], gsems[b])
        pltpu.async_copy(
            u_hbm.at[src_v.at[pl.ds(off, CHUNK)]], rows[b], gsems[b])

    def gather_wait(b):
        pltpu.make_async_copy(
            dst_hbm.at[pl.ds(0, CHUNK)], dstb[b], gsems[b]).wait()
        pltpu.make_async_copy(
            u_hbm.at[src_v.at[pl.ds(0, CHUNK)]], rows[b], gsems[b]).wait()

    def scatter_start(b):
        pltpu.async_copy(rows[b], acc_sh.at[dstb[b]], ssems[b], add=True)

    def scatter_wait(b):
        pltpu.make_async_copy(rows[b], acc_sh.at[dstb[b]], ssems[b]).wait()

    gather_start(0, 0)
    gather_wait(0)
    scatter_start(0)
    gather_start(1, 1)
    gather_wait(1)
    scatter_start(1)
    scatter_wait(0)
    gather_start(2, 0)

    def edge_body(g, _):
        j = 2 * g
        gather_wait(0)
        scatter_start(0)
        scatter_wait(1)
        gather_start(j + 1, 1)
        gather_wait(1)
        scatter_start(1)
        scatter_wait(0)
        gather_start(j + 2, 0)
        return 0

    lax.fori_loop(1, NCHUNK // 2 - 1, edge_body, 0)
    gather_wait(0)
    scatter_start(0)
    scatter_wait(1)
    gather_start(NCHUNK - 1, 1)
    gather_wait(1)
    scatter_start(1)
    scatter_wait(0)
    scatter_wait(1)
    plsc.subcore_barrier()

    for i in range(RPT // 128):
        pltpu.sync_copy(acc_sh.at[pl.ds(s * RPT + i * 128, 128)], buf_v)
        pltpu.sync_copy(buf_v, out_hbm.at[c, pl.ds(s * RPT + i * 128, 128)])


_BLK = 2048
_GRID = N_PAD // _BLK


def _dinv(degp_ref):
    deg = jnp.sum(degp_ref[...], axis=0) + 1.0
    return lax.rsqrt(deg)


def _first_body(x_ref, w_ref, degp_ref, out_ref):
    dinv = _dinv(degp_ref)
    out_ref[...] = jnp.dot(x_ref[...], w_ref[...],
                           preferred_element_type=jnp.float32) * dinv


def _mid_body(agg_ref, u_ref, b_ref, w_ref, degp_ref, out_ref):
    dinv = _dinv(degp_ref)
    t = agg_ref[0] + agg_ref[1] + u_ref[...]
    h = jnp.maximum(t * dinv + b_ref[...], 0.0)
    out_ref[...] = jnp.dot(h, w_ref[...],
                           preferred_element_type=jnp.float32) * dinv


def _last_body(agg_ref, u_ref, b_ref, degp_ref, out_ref):
    i = pl.program_id(0)
    dinv = _dinv(degp_ref)
    t = agg_ref[0] + agg_ref[1] + u_ref[...]
    h = jnp.maximum(t * dinv + b_ref[...], 0.0)
    row = lax.broadcasted_iota(jnp.int32, (_BLK, 1), 0) + i * _BLK
    h = jnp.where(row < N, h, 0.0)
    part = jnp.sum(h, axis=0, keepdims=True) * (1.0 / N)

    @pl.when(i == 0)
    def _():
        out_ref[...] = jnp.zeros_like(out_ref)

    out_ref[...] += part


_rows_spec = pl.BlockSpec((_BLK, D), lambda i: (i, 0))
_w_spec = pl.BlockSpec((D, D), lambda i: (0, 0))
_b_spec = pl.BlockSpec((1, D), lambda i: (0, 0))
_degp_spec = pl.BlockSpec((NC, _BLK, 1), lambda i: (0, i, 0))
_agg_spec = pl.BlockSpec((NC, _BLK, D), lambda i: (0, i, 0))

_first_tc = pl.pallas_call(
    _first_body,
    grid=(_GRID,),
    in_specs=[_rows_spec, _w_spec, _degp_spec],
    out_specs=_rows_spec,
    out_shape=jax.ShapeDtypeStruct((N_PAD, D), jnp.float32),
)

_mid_tc = pl.pallas_call(
    _mid_body,
    grid=(_GRID,),
    in_specs=[_agg_spec, _rows_spec, _b_spec, _w_spec, _degp_spec],
    out_specs=_rows_spec,
    out_shape=jax.ShapeDtypeStruct((N_PAD, D), jnp.float32),
)

_last_tc = pl.pallas_call(
    _last_body,
    grid=(_GRID,),
    in_specs=[_agg_spec, _rows_spec, _b_spec, _degp_spec],
    out_specs=pl.BlockSpec((1, D), lambda i: (0, 0)),
    out_shape=jax.ShapeDtypeStruct((1, D), jnp.float32),
)


def kernel(x, edge_index, W1, b1, W2, b2, W3, b3):
    src = edge_index[0]
    dst = edge_index[1]
    pad_dst = (jnp.arange(NW * (EPP - EPW), dtype=dst.dtype) % (N_PAD - N)
               ).reshape(NW, EPP - EPW) + N
    src_p = jnp.pad(src.reshape(NW, EPW), ((0, 0), (0, EPP - EPW))).reshape(-1)
    dst_p = jnp.concatenate([dst.reshape(NW, EPW), pad_dst], axis=1).reshape(-1)
    x_pad = jnp.pad(x, ((0, N_PAD - N), (0, 0)))
    b1r = b1.reshape(1, D)
    b2r = b2.reshape(1, D)
    b3r = b3.reshape(1, D)

    degp = _deg_kernel(dst)
    degp3 = degp.reshape(NC, N_PAD, 1)

    u1 = _first_tc(x_pad, W1, degp3)
    agg1 = _agg_kernel(u1, src_p, dst_p)
    u2 = _mid_tc(agg1, u1, b1r, W2, degp3)
    agg2 = _agg_kernel(u2, src_p, dst_p)
    u3 = _mid_tc(agg2, u2, b2r, W3, degp3)
    agg3 = _agg_kernel(u3, src_p, dst_p)
    out = _last_tc(agg3, u3, b3r, degp3)
    return out.reshape(D)


# speedup vs baseline: 1.4910x; 1.4866x over previous
import functools

import jax
import jax.numpy as jnp
from jax import lax
from jax.experimental import pallas as pl
from jax.experimental.pallas import tpu as pltpu
from jax.experimental.pallas import tpu_sc as plsc

N = 10000
N_PAD = 10240
E = 320000
D = 128

NC = 2
NS = 16
NW = NC * NS
EPW = E // NW
CHUNK = 80
NCHUNK = EPW // CHUNK
RPT = N_PAD // NS

_mesh = plsc.VectorSubcoreMesh(core_axis_name="c", subcore_axis_name="s")
_sc_params = pltpu.CompilerParams(needs_layout_passes=False)


@functools.partial(
    pl.kernel,
    mesh=_mesh,
    out_type=jax.ShapeDtypeStruct((NC, N_PAD), jnp.float32),
    compiler_params=_sc_params,
    scratch_types=[
        pltpu.VMEM((EPW,), jnp.int32),
        pltpu.VMEM((N_PAD,), jnp.float32),
        pltpu.VMEM((NS, RPT), jnp.float32),
        pltpu.VMEM_SHARED((NS, N_PAD), jnp.float32),
        pltpu.SemaphoreType.DMA,
    ],
)
def _deg_kernel(dst_hbm, out_hbm, dst_v, acc_v, red_v, stage_sh, sem):
    c = lax.axis_index("c")
    s = lax.axis_index("s")
    w = c * NS + s

    icp = pltpu.async_copy(dst_hbm.at[pl.ds(w * EPW, EPW)], dst_v, sem)

    zeros16 = jnp.zeros((16,), jnp.float32)
    ones16 = jnp.ones((16,), jnp.float32)

    def zero_body(i, _):
        acc_v[pl.ds(i * 16, 16)] = zeros16
        return 0

    lax.fori_loop(0, N_PAD // 16, zero_body, 0)
    icp.wait()

    def count_body(j, _):
        idx = dst_v[pl.ds(j * 16, 16)]
        plsc.addupdate_scatter(acc_v, [idx], ones16)
        return 0

    lax.fori_loop(0, EPW // 16, count_body, 0)

    pltpu.sync_copy(acc_v, stage_sh.at[s])
    plsc.subcore_barrier()
    pltpu.sync_copy(stage_sh.at[:, pl.ds(s * RPT, RPT)], red_v)

    def red_body(g, _):
        v = zeros16
        for r in range(NS):
            v = v + red_v[r, pl.ds(g * 16, 16)]
        acc_v[pl.ds(g * 16, 16)] = v
        return 0

    lax.fori_loop(0, RPT // 16, red_body, 0)
    pltpu.sync_copy(acc_v.at[pl.ds(0, RPT)], out_hbm.at[c, pl.ds(s * RPT, RPT)])


@functools.partial(
    pl.kernel,
    mesh=_mesh,
    out_type=jax.ShapeDtypeStruct((NC, N_PAD, D), jnp.float32),
    scratch_types=[
        pltpu.VMEM((EPW,), jnp.int32),
        [pltpu.VMEM((CHUNK,), jnp.int32) for _ in range(2)],
        [pltpu.VMEM((CHUNK, D), jnp.float32) for _ in range(2)],
        pltpu.VMEM((128, D), jnp.float32),
        pltpu.VMEM_SHARED((N_PAD, D), jnp.float32),
        pltpu.SemaphoreType.DMA,
        [pltpu.SemaphoreType.DMA for _ in range(2)],
        [pltpu.SemaphoreType.DMA for _ in range(2)],
    ],
)
def _agg_kernel(u_hbm, src_hbm, dst_hbm, out_hbm,
                src_v, dstb, rows, buf_v, acc_sh, isem, gsems, ssems):
    c = lax.axis_index("c")
    s = lax.axis_index("s")
    w = c * NS + s

    icp = pltpu.async_copy(src_hbm.at[pl.ds(w * EPW, EPW)], src_v, isem)

    zeros16 = jnp.zeros((16,), jnp.float32)

    def zbuf_body(i, _):
        for j in range(D // 16):
            buf_v[i, pl.ds(j * 16, 16)] = zeros16
        return 0

    lax.fori_loop(0, 128, zbuf_body, 0)

    for i in range(RPT // 128):
        pltpu.sync_copy(buf_v, acc_sh.at[pl.ds(s * RPT + i * 128, 128)])
    icp.wait()
    plsc.subcore_barrier()

    def gather_start(j, b):
        base = pl.multiple_of(w * EPW + j * CHUNK, 8)
        off = pl.multiple_of(j * CHUNK, 8)
        pltpu.async_copy(dst_hbm.at[pl.ds(base, CHUNK)], dstb[b], gsems[b])
        pltpu.async_copy(
            u_hbm.at[src_v.at[pl.ds(off, CHUNK)]], rows[b], gsems[b])

    def gather_wait(b):
        pltpu.make_async_copy(
            dst_hbm.at[pl.ds(0, CHUNK)], dstb[b], gsems[b]).wait()
        pltpu.make_async_copy(
            u_hbm.at[src_v.at[pl.ds(0, CHUNK)]], rows[b], gsems[b]).wait()

    def scatter_start(b):
        pltpu.async_copy(rows[b], acc_sh.at[dstb[b]], ssems[b], add=True)

    def scatter_wait(b):
        pltpu.make_async_copy(rows[b], acc_sh.at[dstb[b]], ssems[b]).wait()

    gather_start(0, 0)
    gather_wait(0)
    scatter_start(0)
    gather_start(1, 1)
    gather_wait(1)
    scatter_start(1)
    scatter_wait(0)
    gather_start(2, 0)

    def edge_body(g, _):
        j = 2 * g
        gather_wait(0)
        scatter_start(0)
        scatter_wait(1)
        gather_start(j + 1, 1)
        gather_wait(1)
        scatter_start(1)
        scatter_wait(0)
        gather_start(j + 2, 0)
        return 0

    lax.fori_loop(1, (NCHUNK - 1) // 2, edge_body, 0)
    gather_wait(0)
    scatter_start(0)
    scatter_wait(1)
    scatter_wait(0)
    plsc.subcore_barrier()

    for i in range(RPT // 128):
        pltpu.sync_copy(acc_sh.at[pl.ds(s * RPT + i * 128, 128)], buf_v)
        pltpu.sync_copy(buf_v, out_hbm.at[c, pl.ds(s * RPT + i * 128, 128)])


_BLK = 2048
_GRID = N_PAD // _BLK


def _dinv(degp_ref):
    deg = jnp.sum(degp_ref[...], axis=0) + 1.0
    return lax.rsqrt(deg)


def _first_body(x_ref, w_ref, degp_ref, out_ref):
    dinv = _dinv(degp_ref)
    out_ref[...] = jnp.dot(x_ref[...], w_ref[...],
                           preferred_element_type=jnp.float32) * dinv


def _mid_body(agg_ref, u_ref, b_ref, w_ref, degp_ref, out_ref):
    dinv = _dinv(degp_ref)
    t = agg_ref[0] + agg_ref[1] + u_ref[...]
    h = jnp.maximum(t * dinv + b_ref[...], 0.0)
    out_ref[...] = jnp.dot(h, w_ref[...],
                           preferred_element_type=jnp.float32) * dinv


def _last_body(agg_ref, u_ref, b_ref, degp_ref, out_ref):
    i = pl.program_id(0)
    dinv = _dinv(degp_ref)
    t = agg_ref[0] + agg_ref[1] + u_ref[...]
    h = jnp.maximum(t * dinv + b_ref[...], 0.0)
    row = lax.broadcasted_iota(jnp.int32, (_BLK, 1), 0) + i * _BLK
    h = jnp.where(row < N, h, 0.0)
    part = jnp.sum(h, axis=0, keepdims=True) * (1.0 / N)

    @pl.when(i == 0)
    def _():
        out_ref[...] = jnp.zeros_like(out_ref)

    out_ref[...] += part


_rows_spec = pl.BlockSpec((_BLK, D), lambda i: (i, 0))
_w_spec = pl.BlockSpec((D, D), lambda i: (0, 0))
_b_spec = pl.BlockSpec((1, D), lambda i: (0, 0))
_degp_spec = pl.BlockSpec((NC, _BLK, 1), lambda i: (0, i, 0))
_agg_spec = pl.BlockSpec((NC, _BLK, D), lambda i: (0, i, 0))

_first_tc = pl.pallas_call(
    _first_body,
    grid=(_GRID,),
    in_specs=[_rows_spec, _w_spec, _degp_spec],
    out_specs=_rows_spec,
    out_shape=jax.ShapeDtypeStruct((N_PAD, D), jnp.float32),
)

_mid_tc = pl.pallas_call(
    _mid_body,
    grid=(_GRID,),
    in_specs=[_agg_spec, _rows_spec, _b_spec, _w_spec, _degp_spec],
    out_specs=_rows_spec,
    out_shape=jax.ShapeDtypeStruct((N_PAD, D), jnp.float32),
)

_last_tc = pl.pallas_call(
    _last_body,
    grid=(_GRID,),
    in_specs=[_agg_spec, _rows_spec, _b_spec, _degp_spec],
    out_specs=pl.BlockSpec((1, D), lambda i: (0, 0)),
    out_shape=jax.ShapeDtypeStruct((1, D), jnp.float32),
)


def kernel(x, edge_index, W1, b1, W2, b2, W3, b3):
    src = edge_index[0]
    dst = edge_index[1]
    x_pad = jnp.pad(x, ((0, N_PAD - N), (0, 0)))
    b1r = b1.reshape(1, D)
    b2r = b2.reshape(1, D)
    b3r = b3.reshape(1, D)

    degp = _deg_kernel(dst)
    degp3 = degp.reshape(NC, N_PAD, 1)

    u1 = _first_tc(x_pad, W1, degp3)
    agg1 = _agg_kernel(u1, src, dst)
    u2 = _mid_tc(agg1, u1, b1r, W2, degp3)
    agg2 = _agg_kernel(u2, src, dst)
    u3 = _mid_tc(agg2, u2, b2r, W3, degp3)
    agg3 = _agg_kernel(u3, src, dst)
    out = _last_tc(agg3, u3, b3r, degp3)
    return out.reshape(D)


# speedup vs baseline: 1.5070x; 1.0107x over previous
import functools

import jax
import jax.numpy as jnp
from jax import lax
from jax.experimental import pallas as pl
from jax.experimental.pallas import tpu as pltpu
from jax.experimental.pallas import tpu_sc as plsc

N = 10000
N_PAD = 10240
E = 320000
D = 128

NC = 2
NS = 16
NW = NC * NS
EPW = E // NW
CHUNK = 80
NCHUNK = EPW // CHUNK
RPT = N_PAD // NS

_mesh = plsc.VectorSubcoreMesh(core_axis_name="c", subcore_axis_name="s")
_sc_params = pltpu.CompilerParams(needs_layout_passes=False)


@functools.partial(
    pl.kernel,
    mesh=_mesh,
    out_type=jax.ShapeDtypeStruct((NC, N_PAD), jnp.float32),
    compiler_params=_sc_params,
    scratch_types=[
        pltpu.VMEM((EPW,), jnp.int32),
        pltpu.VMEM((N_PAD,), jnp.float32),
        pltpu.VMEM((NS, RPT), jnp.float32),
        pltpu.VMEM_SHARED((NS, N_PAD), jnp.float32),
        pltpu.SemaphoreType.DMA,
    ],
)
def _deg_kernel(dst_hbm, out_hbm, dst_v, acc_v, red_v, stage_sh, sem):
    c = lax.axis_index("c")
    s = lax.axis_index("s")
    w = c * NS + s

    icp = pltpu.async_copy(dst_hbm.at[pl.ds(w * EPW, EPW)], dst_v, sem)

    zeros16 = jnp.zeros((16,), jnp.float32)
    ones16 = jnp.ones((16,), jnp.float32)

    def zero_body(i, _):
        acc_v[pl.ds(i * 16, 16)] = zeros16
        return 0

    lax.fori_loop(0, N_PAD // 16, zero_body, 0)
    icp.wait()

    def count_body(j, _):
        idx = dst_v[pl.ds(j * 16, 16)]
        plsc.addupdate_scatter(acc_v, [idx], ones16)
        return 0

    lax.fori_loop(0, EPW // 16, count_body, 0)

    pltpu.sync_copy(acc_v, stage_sh.at[s])
    plsc.subcore_barrier()
    pltpu.sync_copy(stage_sh.at[:, pl.ds(s * RPT, RPT)], red_v)

    def red_body(g, _):
        v = zeros16
        for r in range(NS):
            v = v + red_v[r, pl.ds(g * 16, 16)]
        acc_v[pl.ds(g * 16, 16)] = v
        return 0

    lax.fori_loop(0, RPT // 16, red_body, 0)
    pltpu.sync_copy(acc_v.at[pl.ds(0, RPT)], out_hbm.at[c, pl.ds(s * RPT, RPT)])


@functools.partial(
    pl.kernel,
    mesh=_mesh,
    out_type=jax.ShapeDtypeStruct((NC, N_PAD, D), jnp.float32),
    scratch_types=[
        pltpu.VMEM((EPW,), jnp.int32),
        [pltpu.VMEM((CHUNK,), jnp.int32) for _ in range(2)],
        [pltpu.VMEM((CHUNK, D), jnp.float32) for _ in range(2)],
        pltpu.VMEM((128, D), jnp.float32),
        pltpu.VMEM_SHARED((N_PAD, D), jnp.float32),
        pltpu.SemaphoreType.DMA,
        pltpu.SemaphoreType.DMA,
        [pltpu.SemaphoreType.DMA for _ in range(2)],
        [pltpu.SemaphoreType.DMA for _ in range(2)],
    ],
)
def _agg_kernel(u_hbm, src_hbm, dst_hbm, out_hbm,
                src_v, dstb, rows, buf_v, acc_sh, isem, zsem, gsems, ssems):
    c = lax.axis_index("c")
    s = lax.axis_index("s")
    w = c * NS + s

    icp = pltpu.async_copy(src_hbm.at[pl.ds(w * EPW, EPW)], src_v, isem)

    zeros16 = jnp.zeros((16,), jnp.float32)

    def zbuf_body(i, _):
        for j in range(D // 16):
            buf_v[i, pl.ds(j * 16, 16)] = zeros16
        return 0

    lax.fori_loop(0, 128, zbuf_body, 0)

    zcps = [
        pltpu.async_copy(buf_v, acc_sh.at[pl.ds(s * RPT + i * 128, 128)], zsem)
        for i in range(RPT // 128)
    ]
    for zcp in zcps:
        zcp.wait()
    icp.wait()
    plsc.subcore_barrier()

    def gather_start(j, b):
        base = pl.multiple_of(w * EPW + j * CHUNK, 8)
        off = pl.multiple_of(j * CHUNK, 8)
        pltpu.async_copy(dst_hbm.at[pl.ds(base, CHUNK)], dstb[b], gsems[b])
        pltpu.async_copy(
            u_hbm.at[src_v.at[pl.ds(off, CHUNK)]], rows[b], gsems[b])

    def gather_wait(b):
        pltpu.make_async_copy(
            dst_hbm.at[pl.ds(0, CHUNK)], dstb[b], gsems[b]).wait()
        pltpu.make_async_copy(
            u_hbm.at[src_v.at[pl.ds(0, CHUNK)]], rows[b], gsems[b]).wait()

    def scatter_start(b):
        pltpu.async_copy(rows[b], acc_sh.at[dstb[b]], ssems[b], add=True)

    def scatter_wait(b):
        pltpu.make_async_copy(rows[b], acc_sh.at[dstb[b]], ssems[b]).wait()

    gather_start(0, 0)
    gather_wait(0)
    scatter_start(0)
    gather_start(1, 1)
    gather_wait(1)
    scatter_start(1)
    scatter_wait(0)
    gather_start(2, 0)

    def edge_body(g, _):
        j = 2 * g
        gather_wait(0)
        scatter_start(0)
        scatter_wait(1)
        gather_start(j + 1, 1)
        gather_wait(1)
        scatter_start(1)
        scatter_wait(0)
        gather_start(j + 2, 0)
        return 0

    lax.fori_loop(1, (NCHUNK - 1) // 2, edge_body, 0)
    gather_wait(0)
    scatter_start(0)
    scatter_wait(1)
    scatter_wait(0)
    plsc.subcore_barrier()

    wcps = [None, None]
    for i in range(RPT // CHUNK):
        b = i % 2
        if wcps[b] is not None:
            wcps[b].wait()
        pltpu.sync_copy(acc_sh.at[pl.ds(s * RPT + i * CHUNK, CHUNK)], rows[b])
        wcps[b] = pltpu.async_copy(
            rows[b], out_hbm.at[c, pl.ds(s * RPT + i * CHUNK, CHUNK)], gsems[b])
    wcps[0].wait()
    wcps[1].wait()


_BLK = 2048
_GRID = N_PAD // _BLK


def _dinv(degp_ref):
    deg = jnp.sum(degp_ref[...], axis=0) + 1.0
    return lax.rsqrt(deg)


def _first_body(x_ref, w_ref, degp_ref, out_ref):
    dinv = _dinv(degp_ref)
    out_ref[...] = jnp.dot(x_ref[...], w_ref[...],
                           preferred_element_type=jnp.float32) * dinv


def _mid_body(agg_ref, u_ref, b_ref, w_ref, degp_ref, out_ref):
    dinv = _dinv(degp_ref)
    t = agg_ref[0] + agg_ref[1] + u_ref[...]
    h = jnp.maximum(t * dinv + b_ref[...], 0.0)
    out_ref[...] = jnp.dot(h, w_ref[...],
                           preferred_element_type=jnp.float32) * dinv


def _last_body(agg_ref, u_ref, b_ref, degp_ref, out_ref):
    i = pl.program_id(0)
    dinv = _dinv(degp_ref)
    t = agg_ref[0] + agg_ref[1] + u_ref[...]
    h = jnp.maximum(t * dinv + b_ref[...], 0.0)
    row = lax.broadcasted_iota(jnp.int32, (_BLK, 1), 0) + i * _BLK
    h = jnp.where(row < N, h, 0.0)
    part = jnp.sum(h, axis=0, keepdims=True) * (1.0 / N)

    @pl.when(i == 0)
    def _():
        out_ref[...] = jnp.zeros_like(out_ref)

    out_ref[...] += part


_rows_spec = pl.BlockSpec((_BLK, D), lambda i: (i, 0))
_w_spec = pl.BlockSpec((D, D), lambda i: (0, 0))
_b_spec = pl.BlockSpec((1, D), lambda i: (0, 0))
_degp_spec = pl.BlockSpec((NC, _BLK, 1), lambda i: (0, i, 0))
_agg_spec = pl.BlockSpec((NC, _BLK, D), lambda i: (0, i, 0))

_first_tc = pl.pallas_call(
    _first_body,
    grid=(_GRID,),
    in_specs=[_rows_spec, _w_spec, _degp_spec],
    out_specs=_rows_spec,
    out_shape=jax.ShapeDtypeStruct((N_PAD, D), jnp.float32),
)

_mid_tc = pl.pallas_call(
    _mid_body,
    grid=(_GRID,),
    in_specs=[_agg_spec, _rows_spec, _b_spec, _w_spec, _degp_spec],
    out_specs=_rows_spec,
    out_shape=jax.ShapeDtypeStruct((N_PAD, D), jnp.float32),
)

_last_tc = pl.pallas_call(
    _last_body,
    grid=(_GRID,),
    in_specs=[_agg_spec, _rows_spec, _b_spec, _degp_spec],
    out_specs=pl.BlockSpec((1, D), lambda i: (0, 0)),
    out_shape=jax.ShapeDtypeStruct((1, D), jnp.float32),
)


def kernel(x, edge_index, W1, b1, W2, b2, W3, b3):
    src = edge_index[0]
    dst = edge_index[1]
    x_pad = jnp.pad(x, ((0, N_PAD - N), (0, 0)))
    b1r = b1.reshape(1, D)
    b2r = b2.reshape(1, D)
    b3r = b3.reshape(1, D)

    degp = _deg_kernel(dst)
    degp3 = degp.reshape(NC, N_PAD, 1)

    u1 = _first_tc(x_pad, W1, degp3)
    agg1 = _agg_kernel(u1, src, dst)
    u2 = _mid_tc(agg1, u1, b1r, W2, degp3)
    agg2 = _agg_kernel(u2, src, dst)
    u3 = _mid_tc(agg2, u2, b2r, W3, degp3)
    agg3 = _agg_kernel(u3, src, dst)
    out = _last_tc(agg3, u3, b3r, degp3)
    return out.reshape(D)
